# baseline (speedup 1.0000x reference)
import functools
import jax
import jax.numpy as jnp
from jax import lax
from jax.experimental import pallas as pl
from jax.experimental.pallas import tpu as pltpu
from jax.experimental.pallas import tpu_sc as plsc

N = 50000
E = 300000
HID = 128
NH = 8
DH = 16
OUTD = 349
NBLK = 1000
EBLK = 1000
PACK = 144
NPAD = 50176


def _dense_a_body(xp, xa, wip, bip, wia, bia, wq, bq, wkw, bkw, wvw, bvw,
                  wkc, bkc, wvc, bvc, hp_o, q_o, kw_o, vw_o, kc_o, vc_o):
    f32 = jnp.float32
    hp = jnp.maximum(jnp.dot(xp[...], wip[...], preferred_element_type=f32) + bip[...], 0.0)
    ha = jnp.maximum(jnp.dot(xa[...], wia[...], preferred_element_type=f32) + bia[...], 0.0)
    hp_o[...] = hp
    q_o[...] = jnp.dot(hp, wq[...], preferred_element_type=f32) + bq[...]
    kc_o[...] = jnp.dot(hp, wkc[...], preferred_element_type=f32) + bkc[...]
    vc_o[...] = jnp.dot(hp, wvc[...], preferred_element_type=f32) + bvc[...]
    kw_o[...] = jnp.dot(ha, wkw[...], preferred_element_type=f32) + bkw[...]
    vw_o[...] = jnp.dot(ha, wvw[...], preferred_element_type=f32) + bvw[...]


def _edge_b_body(ke, qe, ve, dst, smat, tmat, wv_o, ws_o):
    f32 = jnp.float32
    al = jnp.dot(ke[...] * qe[...], smat[...], preferred_element_type=f32)
    w = jnp.exp(al)
    wexp = jnp.dot(w, smat[...].T, preferred_element_type=f32)
    wv_o[...] = ve[...] * wexp
    wt = jnp.dot(w, tmat[...], preferred_element_type=f32)
    win = lax.broadcasted_iota(jnp.int32, wt.shape, 1) // NH
    m16 = lax.rem(dst[...], 16)
    ws_o[...] = wt * (win == m16).astype(f32)


def _final_c_body(accw, sw, accc, sc_, hp, smat, wa, ba, wout, bout, beta, out):
    f32 = jnp.float32

    def seg(acc, s):
        sexp = jnp.dot(s[...], smat[...].T, preferred_element_type=f32)
        return acc[...] / (sexp + 1e-16)

    agg = seg(accw, sw) + seg(accc, sc_)
    g = 0.5 * agg * (1.0 + lax.erf(agg * 0.7071067811865476))
    o = jnp.dot(g, wa[...], preferred_element_type=f32) + ba[...]
    b = beta[0, 0]
    h2 = b * o + (1.0 - b) * hp[...]
    out[...] = jnp.dot(h2, wout[...], preferred_element_type=f32) + bout[...]


def _rowspec(blk, width):
    return pl.BlockSpec((blk, width), lambda i: (i, 0))


def _fullspec(shape):
    return pl.BlockSpec(shape, lambda i: tuple(0 for _ in shape))


def _dense_a(xp, xa, ws):
    n = xp.shape[0]
    grid = (n // NBLK,)
    outs = [jax.ShapeDtypeStruct((n, HID), jnp.float32)] * 6
    specs = [_rowspec(NBLK, HID)] * 2
    for w in ws:
        specs.append(_fullspec(w.shape))
    return pl.pallas_call(
        _dense_a_body,
        grid=grid,
        in_specs=specs,
        out_specs=[_rowspec(NBLK, HID)] * 6,
        out_shape=outs,
    )(xp, xa, *ws)


def _edge_b(ke, qe, ve, dst, smat, tmat):
    grid = (E // EBLK,)
    return pl.pallas_call(
        _edge_b_body,
        grid=grid,
        in_specs=[_rowspec(EBLK, HID)] * 3 + [_rowspec(EBLK, 1)]
        + [_fullspec(smat.shape), _fullspec(tmat.shape)],
        out_specs=[_rowspec(EBLK, HID)] * 2,
        out_shape=[jax.ShapeDtypeStruct((E, HID), jnp.float32)] * 2,
    )(ke, qe, ve, dst, smat, tmat)


def _final_c(accw, sw, accc, sc_, hp, smat, wa, ba, wout, bout, beta):
    grid = (N // NBLK,)
    return pl.pallas_call(
        _final_c_body,
        grid=grid,
        in_specs=[_rowspec(NBLK, HID), _rowspec(NBLK, NH)] * 2
        + [_rowspec(NBLK, HID)]
        + [_fullspec(smat.shape), _fullspec(wa.shape), _fullspec(ba.shape),
           _fullspec(wout.shape), _fullspec(bout.shape), _fullspec((1, 1))],
        out_specs=_rowspec(NBLK, OUTD),
        out_shape=jax.ShapeDtypeStruct((N, OUTD), jnp.float32),
    )(accw, sw, accc, sc_, hp, smat, wa, ba, wout, bout, beta)


def _fold_kv(wk, bk, wrel, scale):
    wk4 = wk.reshape(HID, NH, DH)
    bk4 = bk.reshape(NH, DH)
    wf = jnp.einsum('nhd,hdf->nhf', wk4, wrel)
    bf = jnp.einsum('hd,hdf->hf', bk4, wrel)
    if scale is not None:
        wf = wf * scale[None, :, None]
        bf = bf * scale[:, None]
    return wf.reshape(HID, HID), bf.reshape(1, HID)


_SC_G = 600
_SC_NCH = E // _SC_G
_SC_NW = 32


def _sc_gather6(kw, q, vw, kc, vc, sw, dw, sci, dc):
    mesh = plsc.VectorSubcoreMesh(core_axis_name="c", subcore_axis_name="s")
    out_t = [jax.ShapeDtypeStruct((E, HID), jnp.float32)] * 6

    @functools.partial(
        pl.kernel, mesh=mesh, out_type=out_t,
        scratch_types=[pltpu.VMEM((_SC_G,), jnp.int32),
                       pltpu.VMEM((_SC_G, HID), jnp.float32),
                       pltpu.SemaphoreType.DMA],
    )
    def body(kw_h, q_h, vw_h, kc_h, vc_h, sw_h, dw_h, sci_h, dc_h,
             o_kw, o_qw, o_vw, o_kc, o_qc, o_vc, idx_v, rows_v, sem):
        wid = lax.axis_index("s") * 2 + lax.axis_index("c")

        def do_chunk(chunk):
            base = chunk * _SC_G
            for idx_h, pairs in ((sw_h, ((kw_h, o_kw), (vw_h, o_vw))),
                                 (dw_h, ((q_h, o_qw),)),
                                 (sci_h, ((kc_h, o_kc), (vc_h, o_vc))),
                                 (dc_h, ((q_h, o_qc),))):
                pltpu.sync_copy(idx_h.at[pl.ds(base, _SC_G)], idx_v)
                for tab, out in pairs:
                    pltpu.async_copy(tab.at[idx_v], rows_v, sem).wait()
                    pltpu.sync_copy(rows_v, out.at[pl.ds(base, _SC_G)])

        def loop_body(j, c):
            chunk = wid + j * _SC_NW

            @pl.when(chunk < _SC_NCH)
            def _():
                do_chunk(chunk)

            return c

        lax.fori_loop(0, (_SC_NCH + _SC_NW - 1) // _SC_NW, loop_body, 0)

    return body(kw, q, vw, kc, vc, sw, dw, sci, dc)


_SB = 96
_SC_NB = E // _SB
_RNG = 12544
_SRNG = _RNG // 16
_ZROWS = 112


def _sc_scatter2(wv_w, ws_w, dst_w, wv_c, ws_c, dst_c):
    mesh = plsc.VectorSubcoreMesh(core_axis_name="c", subcore_axis_name="s")

    @functools.partial(
        pl.kernel, mesh=mesh,
        out_type=[jax.ShapeDtypeStruct((NPAD, HID), jnp.float32)] * 2,
        scratch_types=[pltpu.VMEM((_SB,), jnp.int32),
                       pltpu.VMEM((_SB,), jnp.int32),
                       pltpu.VMEM((_SB, HID), jnp.float32),
                       pltpu.VMEM((_ZROWS, HID), jnp.float32),
                       pltpu.VMEM_SHARED((_RNG + 8, HID), jnp.float32)],
    )
    def body_wv(wvw_h, dw_h, wvc_h, dc_h, ow_h, oc_h,
                dstv, ldstv, wvbuf, zbuf, accs):
        core = lax.axis_index("c")
        sub = lax.axis_index("s")

        def zb_body(r, c):
            for k in range(HID // 16):
                zbuf[r, pl.ds(k * 16, 16)] = jnp.zeros((16,), jnp.float32)
            return c

        lax.fori_loop(0, _ZROWS, zb_body, 0)

        for wv_h, d_h, out_h in ((wvw_h, dw_h, ow_h), (wvc_h, dc_h, oc_h)):
            for p in range(2):
                base = (2 * p + core) * _RNG
                row0 = sub * 784
                for i in range(7):
                    pltpu.sync_copy(zbuf, accs.at[pl.ds(row0 + i * _ZROWS, _ZROWS)])
                plsc.subcore_barrier()

                def bat_body(j, c):
                    b = sub + j * 16

                    @pl.when(b < _SC_NB)
                    def _():
                        e0 = b * _SB
                        pltpu.sync_copy(d_h.at[pl.ds(e0, _SB)], dstv)
                        for k in range(_SB // 16):
                            d = dstv[pl.ds(k * 16, 16)]
                            m = (d >= base) & (d < base + _RNG)
                            ldstv[pl.ds(k * 16, 16)] = jnp.where(m, d - base, _RNG)
                        pltpu.sync_copy(wv_h.at[pl.ds(e0, _SB)], wvbuf)
                        pltpu.sync_copy(wvbuf, accs.at[ldstv], add=True)

                    return c

                lax.fori_loop(0, (_SC_NB + 15) // 16, bat_body, 0)
                plsc.subcore_barrier()
                pltpu.sync_copy(accs.at[pl.ds(row0, 784)],
                                out_h.at[pl.ds(base + row0, 784)])
                plsc.subcore_barrier()

    _SROWS = NPAD // 16
    _SHALF = _SROWS // 2

    @functools.partial(
        pl.kernel, mesh=mesh,
        out_type=[jax.ShapeDtypeStruct((_SROWS, HID), jnp.float32)] * 2,
        scratch_types=[pltpu.VMEM((_SB,), jnp.int32),
                       pltpu.VMEM((_SB,), jnp.int32),
                       pltpu.VMEM((_SB, HID), jnp.float32),
                       pltpu.VMEM((_ZROWS, HID), jnp.float32),
                       pltpu.VMEM_SHARED((_SHALF + 8, HID), jnp.float32)],
    )
    def body_ws(wsw_h, dw_h, wsc_h, dc_h, osw_h, osc_h,
                dstv, ldstv, wsbuf, zbuf, accs):
        core = lax.axis_index("c")
        sub = lax.axis_index("s")

        def zb_body(r, c):
            for k in range(HID // 16):
                zbuf[r, pl.ds(k * 16, 16)] = jnp.zeros((16,), jnp.float32)
            return c

        lax.fori_loop(0, _ZROWS, zb_body, 0)

        for ws_h, d_h, outs_h in ((wsw_h, dw_h, osw_h), (wsc_h, dc_h, osc_h)):
            sbase = core * _SHALF
            srow0 = sub * 98

            @pl.when(sub < 7)
            def _():
                pltpu.sync_copy(zbuf, accs.at[pl.ds(sub * 224, _ZROWS)])
                pltpu.sync_copy(zbuf, accs.at[pl.ds(sub * 224 + _ZROWS, _ZROWS)])

            plsc.subcore_barrier()

            def bat_body(j, c):
                b = sub + j * 16

                @pl.when(b < _SC_NB)
                def _():
                    e0 = b * _SB
                    pltpu.sync_copy(d_h.at[pl.ds(e0, _SB)], dstv)
                    for k in range(_SB // 16):
                        r = lax.shift_right_logical(dstv[pl.ds(k * 16, 16)], 4)
                        m = (r >= sbase) & (r < sbase + _SHALF)
                        ldstv[pl.ds(k * 16, 16)] = jnp.where(m, r - sbase, _SHALF)
                    pltpu.sync_copy(ws_h.at[pl.ds(e0, _SB)], wsbuf)
                    pltpu.sync_copy(wsbuf, accs.at[ldstv], add=True)

                return c

            lax.fori_loop(0, (_SC_NB + 15) // 16, bat_body, 0)
            plsc.subcore_barrier()

            @pl.when(sub < 7)
            def _():
                pltpu.sync_copy(accs.at[pl.ds(sub * 224, 224)],
                                outs_h.at[pl.ds(sbase + sub * 224, 224)])

            plsc.subcore_barrier()

    acc_w, acc_c = body_wv(wv_w, dst_w, wv_c, dst_c)
    sp_w, sp_c = body_ws(ws_w, dst_w, ws_c, dst_c)
    return acc_w, sp_w, acc_c, sp_c


def kernel(x_paper, x_author, params, edge_index_writes, edge_index_cites):
    p = params
    scale_w = p['mu_writes'] * 0.25
    scale_c = p['mu_cites'] * 0.25
    wkw, bkw = _fold_kv(p['Wk_author'], p['bk_author'], p['Watt_writes'], scale_w)
    wvw, bvw = _fold_kv(p['Wv_author'], p['bv_author'], p['Wmsg_writes'], None)
    wkc, bkc = _fold_kv(p['Wk_paper'], p['bk_paper'], p['Watt_cites'], scale_c)
    wvc, bvc = _fold_kv(p['Wv_paper'], p['bv_paper'], p['Wmsg_cites'], None)
    ws = [p['W_in_paper'], p['b_in_paper'].reshape(1, HID),
          p['W_in_author'], p['b_in_author'].reshape(1, HID),
          p['Wq_paper'], p['bq_paper'].reshape(1, HID),
          wkw, bkw, wvw, bvw, wkc, bkc, wvc, bvc]
    hp, q, kw, vw, kc, vc = _dense_a(x_paper, x_author, ws)

    smat = (jnp.arange(HID)[:, None] // DH == jnp.arange(NH)[None, :]).astype(jnp.float32)

    src_w, dst_w = edge_index_writes[0], edge_index_writes[1]
    src_c, dst_c = edge_index_cites[0], edge_index_cites[1]

    tmat = (jnp.arange(NH)[:, None] == jnp.arange(HID)[None, :] % NH).astype(jnp.float32)

    kew, qew, vew, kec, qec, vec = _sc_gather6(
        kw, q, vw, kc, vc, src_w, dst_w, src_c, dst_c)
    wv_w, ws_w = _edge_b(kew, qew, vew, dst_w.reshape(E, 1), smat, tmat)
    wv_c, ws_c = _edge_b(kec, qec, vec, dst_c.reshape(E, 1), smat, tmat)

    acc_w, sp_w, acc_c, sp_c = _sc_scatter2(wv_w, ws_w, dst_w, wv_c, ws_c, dst_c)
    s_w = sp_w.reshape(NPAD, NH)[:N]
    s_c = sp_c.reshape(NPAD, NH)[:N]

    beta = jax.nn.sigmoid(p['skip_paper']).reshape(1, 1)
    return _final_c(acc_w[:N], s_w, acc_c[:N], s_c, hp, smat, p['Wa_paper'],
                    p['ba_paper'].reshape(1, HID), p['W_out'],
                    p['b_out'].reshape(1, OUTD), beta)


# speedup vs baseline: 1.0910x; 1.0910x over previous
import functools
import jax
import jax.numpy as jnp
from jax import lax
from jax.experimental import pallas as pl
from jax.experimental.pallas import tpu as pltpu
from jax.experimental.pallas import tpu_sc as plsc

N = 50000
E = 300000
HID = 128
NH = 8
DH = 16
OUTD = 349
NBLK = 1000
EBLK = 1000
PACK = 144
NPAD = 50176


def _dense_a_body(xp, xa, wip, bip, wia, bia, wq, bq, wkw, bkw, wvw, bvw,
                  wkc, bkc, wvc, bvc, hp_o, q_o, kw_o, vw_o, kc_o, vc_o):
    f32 = jnp.float32
    hp = jnp.maximum(jnp.dot(xp[...], wip[...], preferred_element_type=f32) + bip[...], 0.0)
    ha = jnp.maximum(jnp.dot(xa[...], wia[...], preferred_element_type=f32) + bia[...], 0.0)
    hp_o[...] = hp
    q_o[...] = jnp.dot(hp, wq[...], preferred_element_type=f32) + bq[...]
    kc_o[...] = jnp.dot(hp, wkc[...], preferred_element_type=f32) + bkc[...]
    vc_o[...] = jnp.dot(hp, wvc[...], preferred_element_type=f32) + bvc[...]
    kw_o[...] = jnp.dot(ha, wkw[...], preferred_element_type=f32) + bkw[...]
    vw_o[...] = jnp.dot(ha, wvw[...], preferred_element_type=f32) + bvw[...]


def _edge_b_body(ke, qe, ve, dst, smat, tmat, wv_o, ws_o):
    f32 = jnp.float32
    al = jnp.dot(ke[...] * qe[...], smat[...], preferred_element_type=f32)
    w = jnp.exp(al)
    wexp = jnp.dot(w, smat[...].T, preferred_element_type=f32)
    wv_o[...] = ve[...] * wexp
    wt = jnp.dot(w, tmat[...], preferred_element_type=f32)
    win = lax.broadcasted_iota(jnp.int32, wt.shape, 1) // NH
    m16 = lax.rem(dst[...], 16)
    ws_o[...] = wt * (win == m16).astype(f32)


def _final_c_body(accw, sw, accc, sc_, hp, smat, wa, ba, wout, bout, beta, out):
    f32 = jnp.float32

    def seg(acc, s):
        sexp = jnp.dot(s[...], smat[...].T, preferred_element_type=f32)
        return acc[...] / (sexp + 1e-16)

    agg = seg(accw, sw) + seg(accc, sc_)
    g = 0.5 * agg * (1.0 + lax.erf(agg * 0.7071067811865476))
    o = jnp.dot(g, wa[...], preferred_element_type=f32) + ba[...]
    b = beta[0, 0]
    h2 = b * o + (1.0 - b) * hp[...]
    out[...] = jnp.dot(h2, wout[...], preferred_element_type=f32) + bout[...]


def _rowspec(blk, width):
    return pl.BlockSpec((blk, width), lambda i: (i, 0))


def _fullspec(shape):
    return pl.BlockSpec(shape, lambda i: tuple(0 for _ in shape))


def _dense_a(xp, xa, ws):
    n = xp.shape[0]
    grid = (n // NBLK,)
    outs = [jax.ShapeDtypeStruct((n, HID), jnp.float32)] * 6
    specs = [_rowspec(NBLK, HID)] * 2
    for w in ws:
        specs.append(_fullspec(w.shape))
    return pl.pallas_call(
        _dense_a_body,
        grid=grid,
        in_specs=specs,
        out_specs=[_rowspec(NBLK, HID)] * 6,
        out_shape=outs,
    )(xp, xa, *ws)


def _edge_b(ke, qe, ve, dst, smat, tmat):
    grid = (E // EBLK,)
    return pl.pallas_call(
        _edge_b_body,
        grid=grid,
        in_specs=[_rowspec(EBLK, HID)] * 3 + [_rowspec(EBLK, 1)]
        + [_fullspec(smat.shape), _fullspec(tmat.shape)],
        out_specs=[_rowspec(EBLK, HID)] * 2,
        out_shape=[jax.ShapeDtypeStruct((E, HID), jnp.float32)] * 2,
    )(ke, qe, ve, dst, smat, tmat)


def _final_c(accw, sw, accc, sc_, hp, smat, wa, ba, wout, bout, beta):
    grid = (N // NBLK,)
    return pl.pallas_call(
        _final_c_body,
        grid=grid,
        in_specs=[_rowspec(NBLK, HID), _rowspec(NBLK, NH)] * 2
        + [_rowspec(NBLK, HID)]
        + [_fullspec(smat.shape), _fullspec(wa.shape), _fullspec(ba.shape),
           _fullspec(wout.shape), _fullspec(bout.shape), _fullspec((1, 1))],
        out_specs=_rowspec(NBLK, OUTD),
        out_shape=jax.ShapeDtypeStruct((N, OUTD), jnp.float32),
    )(accw, sw, accc, sc_, hp, smat, wa, ba, wout, bout, beta)


def _fold_kv(wk, bk, wrel, scale):
    wk4 = wk.reshape(HID, NH, DH)
    bk4 = bk.reshape(NH, DH)
    wf = jnp.einsum('nhd,hdf->nhf', wk4, wrel)
    bf = jnp.einsum('hd,hdf->hf', bk4, wrel)
    if scale is not None:
        wf = wf * scale[None, :, None]
        bf = bf * scale[:, None]
    return wf.reshape(HID, HID), bf.reshape(1, HID)


_SC_G = 600
_SC_NCH = E // _SC_G
_SC_NW = 32


def _sc_gather6(kw, q, vw, kc, vc, sw, dw, sci, dc):
    mesh = plsc.VectorSubcoreMesh(core_axis_name="c", subcore_axis_name="s")
    out_t = [jax.ShapeDtypeStruct((E, HID), jnp.float32)] * 6

    @functools.partial(
        pl.kernel, mesh=mesh, out_type=out_t,
        scratch_types=[pltpu.VMEM((_SC_G,), jnp.int32),
                       pltpu.VMEM((_SC_G, HID), jnp.float32),
                       pltpu.SemaphoreType.DMA],
    )
    def body(kw_h, q_h, vw_h, kc_h, vc_h, sw_h, dw_h, sci_h, dc_h,
             o_kw, o_qw, o_vw, o_kc, o_qc, o_vc, idx_v, rows_v, sem):
        wid = lax.axis_index("s") * 2 + lax.axis_index("c")

        def do_chunk(chunk):
            base = chunk * _SC_G
            for idx_h, pairs in ((sw_h, ((kw_h, o_kw), (vw_h, o_vw))),
                                 (dw_h, ((q_h, o_qw),)),
                                 (sci_h, ((kc_h, o_kc), (vc_h, o_vc))),
                                 (dc_h, ((q_h, o_qc),))):
                pltpu.sync_copy(idx_h.at[pl.ds(base, _SC_G)], idx_v)
                for tab, out in pairs:
                    pltpu.async_copy(tab.at[idx_v], rows_v, sem).wait()
                    pltpu.sync_copy(rows_v, out.at[pl.ds(base, _SC_G)])

        def loop_body(j, c):
            chunk = wid + j * _SC_NW

            @pl.when(chunk < _SC_NCH)
            def _():
                do_chunk(chunk)

            return c

        lax.fori_loop(0, (_SC_NCH + _SC_NW - 1) // _SC_NW, loop_body, 0)

    return body(kw, q, vw, kc, vc, sw, dw, sci, dc)


_SB = 96
_SC_NB = E // _SB
_RNG = 12544
_SRNG = _RNG // 16
_ZROWS = 112


def _sc_scatter2(wv_w, ws_w, dst_w, wv_c, ws_c, dst_c):
    mesh = plsc.VectorSubcoreMesh(core_axis_name="c", subcore_axis_name="s")

    def _pipe_scan(sub, d_h, src_h, accs, make_idx, dvs, ldvs, bufs, sems):
        def start(j, par):
            b = sub + j * 16

            @pl.when(b < _SC_NB)
            def _():
                e0 = b * _SB
                pltpu.async_copy(d_h.at[pl.ds(e0, _SB)], dvs[par], sems[par])
                pltpu.async_copy(src_h.at[pl.ds(e0, _SB)], bufs[par], sems[par])

        def proc(j, par):
            b = sub + j * 16

            @pl.when(b < _SC_NB)
            def _():
                e0 = b * _SB
                pltpu.make_async_copy(d_h.at[pl.ds(e0, _SB)], dvs[par], sems[par]).wait()
                pltpu.make_async_copy(src_h.at[pl.ds(e0, _SB)], bufs[par], sems[par]).wait()
                for k in range(_SB // 16):
                    d = dvs[par][pl.ds(k * 16, 16)]
                    ldvs[par][pl.ds(k * 16, 16)] = make_idx(d)
                pltpu.sync_copy(bufs[par], accs.at[ldvs[par]], add=True)

        start(0, 0)
        start(1, 1)

        def bat_body(j2, c):
            j = j2 * 2
            proc(j, 0)
            start(j + 2, 0)
            proc(j + 1, 1)
            start(j + 3, 1)
            return c

        nj2 = ((_SC_NB + 15) // 16 + 1) // 2
        lax.fori_loop(0, nj2, bat_body, 0)

    @functools.partial(
        pl.kernel, mesh=mesh,
        out_type=[jax.ShapeDtypeStruct((NPAD, HID), jnp.float32)] * 2,
        scratch_types=[pltpu.VMEM((_SB,), jnp.int32),
                       pltpu.VMEM((_SB,), jnp.int32),
                       pltpu.VMEM((_SB, HID), jnp.float32),
                       pltpu.VMEM((_ZROWS, HID), jnp.float32),
                       pltpu.VMEM_SHARED((_RNG + 8, HID), jnp.float32)],
    )
    def body_wv(wvw_h, dw_h, wvc_h, dc_h, ow_h, oc_h,
                dstv, ldstv, wvbuf, zbuf, accs):
        core = lax.axis_index("c")
        sub = lax.axis_index("s")

        def zb_body(r, c):
            for k in range(HID // 16):
                zbuf[r, pl.ds(k * 16, 16)] = jnp.zeros((16,), jnp.float32)
            return c

        lax.fori_loop(0, _ZROWS, zb_body, 0)

        for wv_h, d_h, out_h in ((wvw_h, dw_h, ow_h), (wvc_h, dc_h, oc_h)):
            for p in range(2):
                base = (2 * p + core) * _RNG
                row0 = sub * 784
                for i in range(7):
                    pltpu.sync_copy(zbuf, accs.at[pl.ds(row0 + i * _ZROWS, _ZROWS)])
                plsc.subcore_barrier()

                def bat_body(j, c):
                    b = sub + j * 16

                    @pl.when(b < _SC_NB)
                    def _():
                        e0 = b * _SB
                        pltpu.sync_copy(d_h.at[pl.ds(e0, _SB)], dstv)
                        for k in range(_SB // 16):
                            d = dstv[pl.ds(k * 16, 16)]
                            m = (d >= base) & (d < base + _RNG)
                            ldstv[pl.ds(k * 16, 16)] = jnp.where(m, d - base, _RNG)
                        pltpu.sync_copy(wv_h.at[pl.ds(e0, _SB)], wvbuf)
                        pltpu.sync_copy(wvbuf, accs.at[ldstv], add=True)

                    return c

                lax.fori_loop(0, (_SC_NB + 15) // 16, bat_body, 0)
                plsc.subcore_barrier()
                pltpu.sync_copy(accs.at[pl.ds(row0, 784)],
                                out_h.at[pl.ds(base + row0, 784)])
                plsc.subcore_barrier()

    _SROWS = NPAD // 16
    _SHALF = _SROWS // 2

    @functools.partial(
        pl.kernel, mesh=mesh,
        out_type=[jax.ShapeDtypeStruct((_SROWS, HID), jnp.float32)] * 2,
        scratch_types=[pltpu.VMEM((_SB,), jnp.int32),
                       pltpu.VMEM((_SB,), jnp.int32),
                       pltpu.VMEM((_SB,), jnp.int32),
                       pltpu.VMEM((_SB,), jnp.int32),
                       pltpu.VMEM((_SB, HID), jnp.float32),
                       pltpu.VMEM((_SB, HID), jnp.float32),
                       pltpu.VMEM((_ZROWS, HID), jnp.float32),
                       pltpu.VMEM_SHARED((_SHALF + 8, HID), jnp.float32),
                       pltpu.SemaphoreType.DMA,
                       pltpu.SemaphoreType.DMA],
    )
    def body_ws(wsw_h, dw_h, wsc_h, dc_h, osw_h, osc_h,
                dv0, dv1, ldv0, ldv1, buf0, buf1, zbuf, accs, sem0, sem1):
        core = lax.axis_index("c")
        sub = lax.axis_index("s")

        def zb_body(r, c):
            for k in range(HID // 16):
                zbuf[r, pl.ds(k * 16, 16)] = jnp.zeros((16,), jnp.float32)
            return c

        lax.fori_loop(0, _ZROWS, zb_body, 0)

        for ws_h, d_h, outs_h in ((wsw_h, dw_h, osw_h), (wsc_h, dc_h, osc_h)):
            sbase = core * _SHALF

            @pl.when(sub < 7)
            def _():
                pltpu.sync_copy(zbuf, accs.at[pl.ds(sub * 224, _ZROWS)])
                pltpu.sync_copy(zbuf, accs.at[pl.ds(sub * 224 + _ZROWS, _ZROWS)])

            plsc.subcore_barrier()

            def mk_idx(d, sbase=sbase):
                r = lax.shift_right_logical(d, 4)
                m = (r >= sbase) & (r < sbase + _SHALF)
                return jnp.where(m, r - sbase, _SHALF)

            _pipe_scan(sub, d_h, ws_h, accs, mk_idx,
                       (dv0, dv1), (ldv0, ldv1), (buf0, buf1), (sem0, sem1))
            plsc.subcore_barrier()

            @pl.when(sub < 7)
            def _():
                pltpu.sync_copy(accs.at[pl.ds(sub * 224, 224)],
                                outs_h.at[pl.ds(sbase + sub * 224, 224)])

            plsc.subcore_barrier()

    acc_w, acc_c = body_wv(wv_w, dst_w, wv_c, dst_c)
    sp_w, sp_c = body_ws(ws_w, dst_w, ws_c, dst_c)
    return acc_w, sp_w, acc_c, sp_c


def kernel(x_paper, x_author, params, edge_index_writes, edge_index_cites):
    p = params
    scale_w = p['mu_writes'] * 0.25
    scale_c = p['mu_cites'] * 0.25
    wkw, bkw = _fold_kv(p['Wk_author'], p['bk_author'], p['Watt_writes'], scale_w)
    wvw, bvw = _fold_kv(p['Wv_author'], p['bv_author'], p['Wmsg_writes'], None)
    wkc, bkc = _fold_kv(p['Wk_paper'], p['bk_paper'], p['Watt_cites'], scale_c)
    wvc, bvc = _fold_kv(p['Wv_paper'], p['bv_paper'], p['Wmsg_cites'], None)
    ws = [p['W_in_paper'], p['b_in_paper'].reshape(1, HID),
          p['W_in_author'], p['b_in_author'].reshape(1, HID),
          p['Wq_paper'], p['bq_paper'].reshape(1, HID),
          wkw, bkw, wvw, bvw, wkc, bkc, wvc, bvc]
    hp, q, kw, vw, kc, vc = _dense_a(x_paper, x_author, ws)

    smat = (jnp.arange(HID)[:, None] // DH == jnp.arange(NH)[None, :]).astype(jnp.float32)

    src_w, dst_w = edge_index_writes[0], edge_index_writes[1]
    src_c, dst_c = edge_index_cites[0], edge_index_cites[1]

    tmat = (jnp.arange(NH)[:, None] == jnp.arange(HID)[None, :] % NH).astype(jnp.float32)

    kew, qew, vew, kec, qec, vec = _sc_gather6(
        kw, q, vw, kc, vc, src_w, dst_w, src_c, dst_c)
    wv_w, ws_w = _edge_b(kew, qew, vew, dst_w.reshape(E, 1), smat, tmat)
    wv_c, ws_c = _edge_b(kec, qec, vec, dst_c.reshape(E, 1), smat, tmat)

    acc_w, sp_w, acc_c, sp_c = _sc_scatter2(wv_w, ws_w, dst_w, wv_c, ws_c, dst_c)
    s_w = sp_w.reshape(NPAD, NH)[:N]
    s_c = sp_c.reshape(NPAD, NH)[:N]

    beta = jax.nn.sigmoid(p['skip_paper']).reshape(1, 1)
    return _final_c(acc_w[:N], s_w, acc_c[:N], s_c, hp, smat, p['Wa_paper'],
                    p['ba_paper'].reshape(1, HID), p['W_out'],
                    p['b_out'].reshape(1, OUTD), beta)
